# baseline (speedup 1.0000x reference)
import functools

import jax
import jax.numpy as jnp
from jax import lax
from jax.experimental import pallas as pl
from jax.experimental.pallas import tpu as pltpu
from jax.experimental.pallas import tpu_sc as plsc

N = 320000
D = 128
S = 10000
NC = 2
SH = S // NC
NS = 16
CHUNK = 128
NCHUNKS = N // CHUNK
NBUF = 4
AROWS = SH + 8

_mesh = plsc.VectorSubcoreMesh(core_axis_name="c", subcore_axis_name="s")


@functools.partial(
    pl.kernel,
    out_type=jax.ShapeDtypeStruct((S, D), jnp.float32),
    mesh=_mesh,
    scratch_types=[
        pltpu.VMEM((NBUF, CHUNK, D), jnp.float32),
        pltpu.VMEM((NBUF, 128), jnp.int32),
        pltpu.VMEM((32,), jnp.int32),
        pltpu.VMEM_SHARED((AROWS, D), jnp.float32),
        pltpu.SemaphoreType.DMA,
        pltpu.SemaphoreType.DMA,
        pltpu.SemaphoreType.DMA,
        pltpu.SemaphoreType.DMA,
    ],
)
def _seg_sum_sc(feat_hbm, idx_hbm, params_hbm, out_hbm,
                feat_bufs, idx_bufs, pbuf, acc, gsem0, gsem1, gsem2, gsem3):
    c = lax.axis_index("c")
    s = lax.axis_index("s")
    gsems = (gsem0, gsem1, gsem2, gsem3)

    zero16 = jnp.zeros((16,), jnp.float32)

    def zrow(i, carry):
        for j in range(D // 16):
            feat_bufs[0, i, pl.ds(j * 16, 16)] = zero16
        return carry

    lax.fori_loop(0, CHUNK, zrow, 0)
    zbuf = feat_bufs.at[0]
    off = pl.multiple_of(s * (AROWS // NS) - (s % 8), 8)
    for z in range(312 // CHUNK):
        pltpu.sync_copy(zbuf, acc.at[pl.ds(pl.multiple_of(off + z * CHUNK, 8),
                                           CHUNK)])
    zrem = 312 % CHUNK
    if zrem:
        pltpu.sync_copy(zbuf.at[pl.ds(0, zrem)],
                        acc.at[pl.ds(pl.multiple_of(off + 312 - zrem, 8), zrem)])

    @pl.when(s % 8 == 7)
    def _zero_tail():
        pltpu.sync_copy(zbuf.at[pl.ds(0, 8)],
                        acc.at[pl.ds(pl.multiple_of(off + 312, 8), 8)])

    pltpu.sync_copy(params_hbm, pbuf)
    nc0 = pbuf[pl.ds(0, 16)][0]
    nc1 = pbuf[pl.ds(16, 16)][0]
    mybase = c * nc1
    mycnt = (1 - c) * nc0 + c * (NCHUNKS - nc1)
    q = (mycnt + NS - 1) // NS
    lo = s * q
    qs = jnp.minimum(jnp.maximum(mycnt - lo, 0), q)

    plsc.subcore_barrier()

    rowbase = c * SH

    def issue_gather(b, k):
        rbase = pl.multiple_of((mybase + lo + k) * CHUNK, 8)
        pltpu.async_copy(feat_hbm.at[pl.ds(rbase, CHUNK)],
                         feat_bufs.at[b], gsems[b])
        pltpu.async_copy(idx_hbm.at[pl.ds(rbase, CHUNK)],
                         idx_bufs.at[b], gsems[b])

    def wait_gather(b, k):
        rbase = pl.multiple_of((mybase + lo + k) * CHUNK, 8)
        pltpu.make_async_copy(feat_hbm.at[pl.ds(rbase, CHUNK)],
                              feat_bufs.at[b], gsems[b]).wait()
        pltpu.make_async_copy(idx_hbm.at[pl.ds(rbase, CHUNK)],
                              idx_bufs.at[b], gsems[b]).wait()

    for b in range(NBUF):
        @pl.when(b < qs)
        def _prime():
            issue_gather(b, b)

    def superstep(i, carry):
        for b in range(NBUF):
            k = i * NBUF + b

            @pl.when(k < qs)
            def _do():
                wait_gather(b, k)
                for j in range(128 // 16):
                    v = idx_bufs[b, pl.ds(j * 16, 16)] - rowbase
                    oob = (v < 0) | (v >= SH)
                    idx_bufs[b, pl.ds(j * 16, 16)] = jnp.where(oob, SH, v)
                pltpu.sync_copy(feat_bufs.at[b], acc.at[idx_bufs.at[b]],
                                add=True)

                @pl.when(k + NBUF < qs)
                def _prefetch():
                    issue_gather(b, k + NBUF)
        return carry

    lax.fori_loop(0, (q + NBUF - 1) // NBUF, superstep, 0)
    plsc.subcore_barrier()

    obase = pl.multiple_of(c * SH + off, 8)
    pltpu.sync_copy(acc.at[pl.ds(off, 312)], out_hbm.at[pl.ds(obase, 312)])

    @pl.when((s % 8 == 7) & (s < NS - 1))
    def _dump_tail():
        pltpu.sync_copy(acc.at[pl.ds(pl.multiple_of(off + 312, 8), 8)],
                        out_hbm.at[pl.ds(pl.multiple_of(c * SH + off + 312, 8), 8)])


@jax.jit
def kernel(features, structural_indices):
    p = jnp.searchsorted(structural_indices, SH,
                         side="left").astype(jnp.int32)
    nc0 = (p + CHUNK - 1) // CHUNK
    nc1 = p // CHUNK
    params = jnp.concatenate([jnp.full((16,), nc0, jnp.int32),
                              jnp.full((16,), nc1, jnp.int32)])
    return _seg_sum_sc(features, structural_indices, params)


# speedup vs baseline: 1.2768x; 1.2768x over previous
import functools

import jax
import jax.numpy as jnp
from jax import lax
from jax.experimental import pallas as pl
from jax.experimental.pallas import tpu as pltpu
from jax.experimental.pallas import tpu_sc as plsc

N = 320000
D = 128
S = 10000
NC = 2
SH = S // NC
NS = 16
CHUNK = 128
NCHUNKS = N // CHUNK
NBUF = 4
AROWS = SH + 8

_mesh = plsc.VectorSubcoreMesh(core_axis_name="c", subcore_axis_name="s")


@functools.partial(
    pl.kernel,
    out_type=jax.ShapeDtypeStruct((S, D), jnp.float32),
    mesh=_mesh,
    scratch_types=[
        pltpu.VMEM((NBUF, CHUNK, D), jnp.float32),
        pltpu.VMEM((NBUF, 128), jnp.int32),
        pltpu.VMEM((16,), jnp.int32),
        pltpu.VMEM_SHARED((AROWS, D), jnp.float32),
        pltpu.SemaphoreType.DMA,
        pltpu.SemaphoreType.DMA,
        pltpu.SemaphoreType.DMA,
        pltpu.SemaphoreType.DMA,
        pltpu.SemaphoreType.DMA,
    ],
)
def _seg_sum_sc(feat_hbm, idx_hbm, out_hbm,
                feat_bufs, idx_bufs, pbuf, acc,
                gsem0, gsem1, gsem2, gsem3, zsem):
    c = lax.axis_index("c")
    s = lax.axis_index("s")
    gsems = (gsem0, gsem1, gsem2, gsem3)

    zero16 = jnp.zeros((16,), jnp.float32)

    def zrow(i, carry):
        for j in range(D // 16):
            feat_bufs[0, i, pl.ds(j * 16, 16)] = zero16
        return carry

    lax.fori_loop(0, CHUNK, zrow, 0)
    zbuf = feat_bufs.at[0]
    off = pl.multiple_of(s * (AROWS // NS) - (s % 8), 8)
    zcopies = [(zbuf, acc.at[pl.ds(pl.multiple_of(off + z * CHUNK, 8), CHUNK)])
               for z in range(312 // CHUNK)]
    zrem = 312 % CHUNK
    if zrem:
        zcopies.append(
            (zbuf.at[pl.ds(0, zrem)],
             acc.at[pl.ds(pl.multiple_of(off + 312 - zrem, 8), zrem)]))
    for src, dst in zcopies:
        pltpu.async_copy(src, dst, zsem)

    @pl.when(s % 8 == 7)
    def _zero_tail():
        pltpu.async_copy(zbuf.at[pl.ds(0, 8)],
                         acc.at[pl.ds(pl.multiple_of(off + 312, 8), 8)], zsem)

    def probe(_, lohi):
        lo, hi = lohi
        mid = (lo + hi) // 2

        def probed():
            pltpu.sync_copy(idx_hbm.at[pl.ds(pl.multiple_of(mid * CHUNK, 8),
                                             16)], pbuf)
            v = pbuf[...][0]
            ge = (v >= SH).astype(jnp.int32)
            return ge * mid + (1 - ge) * hi, ge * lo + (1 - ge) * (mid + 1)

        def done():
            return hi, lo

        hi2, lo2 = lax.cond(lo < hi, probed, done)
        return lo2, hi2

    lo0, hi0 = jnp.int32(0), jnp.int32(NCHUNKS)
    r_lo, r_hi = lax.fori_loop(0, 12, probe, (lo0, hi0))
    r = r_hi
    nc0 = r
    nc1 = jnp.maximum(r - 1, 0)

    for src, dst in zcopies:
        pltpu.make_async_copy(src, dst, zsem).wait()

    @pl.when(s % 8 == 7)
    def _zero_tail_wait():
        pltpu.make_async_copy(
            zbuf.at[pl.ds(0, 8)],
            acc.at[pl.ds(pl.multiple_of(off + 312, 8), 8)], zsem).wait()
    mybase = c * nc1
    mycnt = (1 - c) * nc0 + c * (NCHUNKS - nc1)
    q = (mycnt + NS - 1) // NS
    lo = s * q
    qs = jnp.minimum(jnp.maximum(mycnt - lo, 0), q)

    plsc.subcore_barrier()

    rowbase = c * SH

    def issue_gather(b, k):
        rbase = pl.multiple_of((mybase + lo + k) * CHUNK, 8)
        pltpu.async_copy(feat_hbm.at[pl.ds(rbase, CHUNK)],
                         feat_bufs.at[b], gsems[b])
        pltpu.async_copy(idx_hbm.at[pl.ds(rbase, CHUNK)],
                         idx_bufs.at[b], gsems[b])

    def wait_gather(b, k):
        rbase = pl.multiple_of((mybase + lo + k) * CHUNK, 8)
        pltpu.make_async_copy(feat_hbm.at[pl.ds(rbase, CHUNK)],
                              feat_bufs.at[b], gsems[b]).wait()
        pltpu.make_async_copy(idx_hbm.at[pl.ds(rbase, CHUNK)],
                              idx_bufs.at[b], gsems[b]).wait()

    for b in range(NBUF):
        @pl.when(b < qs)
        def _prime():
            issue_gather(b, b)

    def superstep(i, carry):
        for b in range(NBUF):
            k = i * NBUF + b

            @pl.when(k < qs)
            def _do():
                wait_gather(b, k)
                for j in range(128 // 16):
                    v = idx_bufs[b, pl.ds(j * 16, 16)] - rowbase
                    oob = (v < 0) | (v >= SH)
                    idx_bufs[b, pl.ds(j * 16, 16)] = jnp.where(oob, SH, v)
                pltpu.sync_copy(feat_bufs.at[b], acc.at[idx_bufs.at[b]],
                                add=True)

                @pl.when(k + NBUF < qs)
                def _prefetch():
                    issue_gather(b, k + NBUF)
        return carry

    lax.fori_loop(0, (q + NBUF - 1) // NBUF, superstep, 0)
    plsc.subcore_barrier()

    obase = pl.multiple_of(c * SH + off, 8)
    pltpu.sync_copy(acc.at[pl.ds(off, 312)], out_hbm.at[pl.ds(obase, 312)])

    @pl.when((s % 8 == 7) & (s < NS - 1))
    def _dump_tail():
        pltpu.sync_copy(acc.at[pl.ds(pl.multiple_of(off + 312, 8), 8)],
                        out_hbm.at[pl.ds(pl.multiple_of(c * SH + off + 312, 8), 8)])


@jax.jit
def kernel(features, structural_indices):
    return _seg_sum_sc(features, structural_indices)
